# baseline (speedup 1.0000x reference)
import functools

import jax
import jax.numpy as jnp
from jax import lax
from jax.experimental import pallas as pl
from jax.experimental.pallas import tpu as pltpu
from jax.experimental.pallas import tpu_sc as plsc

NUM_FEATURES = 22528
L1 = 1024
L2 = 15
L3 = 32
NUM_BUCKETS = 9
MAX_ACTIVE = 32
BATCH = 4096

_NC = 2
_NS = 16
_LANES = 16
_NW = _NC * _NS
_NBAGS = 2 * BATCH
_BPW = _NBAGS // _NW

_TB = 512
_L1P = 144
_WOP = 16


_DW = L1 // 2


def _ft_sparsecore(table_i32, idx_all, val_all):
    mesh = plsc.VectorSubcoreMesh(core_axis_name="c", subcore_axis_name="s")
    cp = pltpu.CompilerParams(needs_layout_passes=False)
    hb = _BPW // 2

    @functools.partial(
        pl.kernel,
        out_type=jax.ShapeDtypeStruct((_NBAGS, L1), jnp.float32),
        mesh=mesh,
        compiler_params=cp,
        scratch_types=[
            pltpu.VMEM((hb // 2, 2 * MAX_ACTIVE), jnp.int32),
            pltpu.VMEM((hb, MAX_ACTIVE), jnp.float32),
            pltpu.VMEM((2 * MAX_ACTIVE, _DW), jnp.int32),
            pltpu.VMEM((2 * MAX_ACTIVE, _DW), jnp.int32),
            pltpu.VMEM((L1,), jnp.float32),
            pltpu.VMEM((L1,), jnp.float32),
            pltpu.SemaphoreType.DMA,
            pltpu.SemaphoreType.DMA,
            pltpu.SemaphoreType.DMA,
            pltpu.SemaphoreType.DMA,
        ],
    )
    def ft_kernel(table_hbm, idx_hbm, val_hbm, out_hbm,
                  idx_v, val_v, rows_a, rows_b, acc_a, acc_b,
                  gsem_a, gsem_b, osem_a, osem_b):
        wid = lax.axis_index("s") * _NC + lax.axis_index("c")
        base = wid * _BPW

        rows = (rows_a, rows_b)
        accs = (acc_a, acc_b)
        gsems = (gsem_a, gsem_b)
        osems = (osem_a, osem_b)

        himask = jnp.full((_LANES,), -65536, jnp.int32)

        for h in range(_BPW // hb):
            hbase = base + h * hb

            pbase = wid * (_BPW // 2) + h * (hb // 2)
            pltpu.sync_copy(idx_hbm.at[pl.ds(pbase, hb // 2)], idx_v)
            pltpu.sync_copy(val_hbm.at[pl.ds(hbase, hb)], val_v)

            pltpu.async_copy(table_hbm.at[idx_v.at[0]], rows_a, gsem_a)
            pltpu.async_copy(table_hbm.at[idx_v.at[1]], rows_b, gsem_b)

            @pl.loop(0, hb // 4)
            def _(g):
                for p in range(2):
                    pair = g * 2 + p
                    rbuf = rows[p]
                    gsem = gsems[p]

                    pltpu.make_async_copy(
                        table_hbm.at[idx_v.at[pair]], rbuf, gsem).wait()

                    for j in range(2):
                        bag = pair * 2 + j
                        abuf = accs[j]
                        osem = osems[j]

                        @pl.when(h * hb + bag >= 2)
                        def _():
                            pltpu.make_async_copy(
                                abuf, out_hbm.at[hbase + bag], osem).wait()

                        row_sel = jnp.full((_LANES,), bag, jnp.int32)
                        splats = []
                        for kk in range(MAX_ACTIVE):
                            sp = plsc.load_gather(
                                val_v,
                                [row_sel,
                                 jnp.full((_LANES,), kk, jnp.int32)])
                            splats.append(
                                plsc.pack(sp, sp,
                                          format=plsc.PackFormat.INTERLEAVED))

                        @pl.loop(0, _DW, step=_LANES)
                        def _(d):
                            acc = [jnp.zeros((2 * _LANES,), jnp.bfloat16)
                                   for _ in range(8)]
                            for kk in range(MAX_ACTIVE):
                                v = rbuf[j * MAX_ACTIVE + kk,
                                         pl.ds(d, _LANES)]
                                q = kk % 8
                                acc[q] = acc[q] + (
                                    plsc.bitcast(v, jnp.bfloat16)
                                    * splats[kk])
                            los = []
                            his = []
                            for q in range(8):
                                pi = plsc.bitcast(acc[q], jnp.int32)
                                los.append(plsc.bitcast(
                                    lax.shift_left(pi, 16), jnp.float32))
                                his.append(
                                    plsc.bitcast(pi & himask, jnp.float32))
                            lo = (((los[0] + los[1]) + (los[2] + los[3]))
                                  + ((los[4] + los[5]) + (los[6] + los[7])))
                            hi = (((his[0] + his[1]) + (his[2] + his[3]))
                                  + ((his[4] + his[5]) + (his[6] + his[7])))
                            abuf[pl.ds(d, _LANES)] = lo
                            abuf[pl.ds(_DW + d, _LANES)] = hi

                        pltpu.async_copy(
                            abuf, out_hbm.at[hbase + bag], osem)

                    @pl.when(pair + 2 < hb // 2)
                    def _():
                        pltpu.async_copy(
                            table_hbm.at[idx_v.at[pair + 2]], rbuf, gsem)

        pltpu.make_async_copy(acc_a, out_hbm.at[base], osem_a).wait()
        pltpu.make_async_copy(acc_b, out_hbm.at[base], osem_b).wait()

    return ft_kernel(table_i32, idx_all, val_all)


def _pack_body(x_ref, o_ref):
    bits = lax.bitcast_convert_type(x_ref[...], jnp.uint32)
    r = bits + jnp.uint32(0x8000)
    word = (r[:, :_DW] >> 16) | (r[:, _DW:] & jnp.uint32(0xFFFF0000))
    o_ref[...] = lax.bitcast_convert_type(word, jnp.int32)


def _pack_tc(ft_weight):
    rb = NUM_FEATURES // 32
    return pl.pallas_call(
        _pack_body,
        grid=(32,),
        in_specs=[pl.BlockSpec((rb, L1), lambda i: (i, 0))],
        out_specs=pl.BlockSpec((rb, _DW), lambda i: (i, 0)),
        out_shape=jax.ShapeDtypeStruct((NUM_FEATURES, _DW), jnp.int32),
    )(ft_weight)


def _trunk_body(w_ref, b_ref, us_ref, them_ref, ls_ref, ftb_ref,
                w1_ref, b1_ref, w2_ref, b2_ref, wo_ref, bo_ref, o_ref):
    w = w_ref[...] + ftb_ref[...]
    b = b_ref[...] + ftb_ref[...]
    us = us_ref[...]
    them = them_ref[...]
    h1 = jnp.clip(us * w + them * b, 0.0, 1.0)
    h2 = jnp.clip(us * b + them * w, 0.0, 1.0)
    s = L1 // 2
    x = jnp.concatenate(
        [h1[:, :s] * h1[:, s:], h2[:, :s] * h2[:, s:]], axis=1)
    x = x * (127.0 / 128.0)

    idx = ls_ref[...]

    dn = (((1,), (1,)), ((), ()))
    l1 = lax.dot_general(x, w1_ref[...], dn,
                         preferred_element_type=jnp.float32) + b1_ref[...]
    c1 = lax.broadcasted_iota(jnp.int32, (_TB, _L1P), 1)
    l1 = jnp.where((c1 // L2) == idx, l1, 0.0)
    p1 = (lax.broadcasted_iota(jnp.int32, (_L1P, L2), 0) % L2
          == lax.broadcasted_iota(jnp.int32, (_L1P, L2), 1)
          ).astype(jnp.float32)
    l1 = lax.dot_general(l1, p1, (((1,), (0,)), ((), ())),
                         preferred_element_type=jnp.float32)
    l1 = jnp.clip(l1, 0.0, 1.0)

    l2 = lax.dot_general(l1, w2_ref[...], dn,
                         preferred_element_type=jnp.float32) + b2_ref[...]
    c2 = lax.broadcasted_iota(jnp.int32, (_TB, L3 * NUM_BUCKETS), 1)
    l2 = jnp.where((c2 // L3) == idx, l2, 0.0)
    p2 = (lax.broadcasted_iota(jnp.int32, (L3 * NUM_BUCKETS, L3), 0) % L3
          == lax.broadcasted_iota(jnp.int32, (L3 * NUM_BUCKETS, L3), 1)
          ).astype(jnp.float32)
    l2 = lax.dot_general(l2, p2, (((1,), (0,)), ((), ())),
                         preferred_element_type=jnp.float32)
    l2 = jnp.clip(l2, 0.0, 1.0)

    o = lax.dot_general(l2, wo_ref[...], dn,
                        preferred_element_type=jnp.float32) + bo_ref[...]
    c3 = lax.broadcasted_iota(jnp.int32, (_TB, _WOP), 1)
    o = jnp.where(c3 == idx, o, 0.0)
    o_ref[...] = jnp.sum(o, axis=1, keepdims=True)


def _trunk_tc(ft_out, us, them, ls_idx, ft_bias, W1, b1, W2, b2, Wo, bo):
    nblk = BATCH // _TB
    w1p = jnp.pad(W1, ((0, _L1P - L2 * NUM_BUCKETS), (0, 0)))
    b1p = jnp.pad(b1, (0, _L1P - L2 * NUM_BUCKETS)).reshape(1, _L1P)
    wop = jnp.pad(Wo, ((0, _WOP - NUM_BUCKETS), (0, 0)))
    bop = jnp.pad(bo, (0, _WOP - NUM_BUCKETS)).reshape(1, _WOP)

    full = lambda shape: pl.BlockSpec(shape, lambda i: (0, 0))
    return pl.pallas_call(
        _trunk_body,
        grid=(nblk,),
        in_specs=[
            pl.BlockSpec((_TB, L1), lambda i: (i, 0)),
            pl.BlockSpec((_TB, L1), lambda i: (i + nblk, 0)),
            pl.BlockSpec((_TB, 1), lambda i: (i, 0)),
            pl.BlockSpec((_TB, 1), lambda i: (i, 0)),
            pl.BlockSpec((_TB, 1), lambda i: (i, 0)),
            full((1, L1)),
            full((_L1P, L1)),
            full((1, _L1P)),
            full((L3 * NUM_BUCKETS, L2)),
            full((1, L3 * NUM_BUCKETS)),
            full((_WOP, L3)),
            full((1, _WOP)),
        ],
        out_specs=pl.BlockSpec((_TB, 1), lambda i: (i, 0)),
        out_shape=jax.ShapeDtypeStruct((BATCH, 1), jnp.float32),
    )(ft_out, ft_out, us, them, ls_idx, ft_bias.reshape(1, L1),
      w1p, b1p, W2, b2.reshape(1, L3 * NUM_BUCKETS), wop, bop)


def kernel(us, them, white_indices, white_values, black_indices, black_values,
           layer_stack_indices, ft_weight, ft_bias, W1, b1, W2, b2, Wo, bo):
    idx_all = jnp.concatenate([white_indices, black_indices], axis=0)
    idx_all = idx_all.reshape(_NBAGS // 2, 2 * MAX_ACTIVE)
    val_all = jnp.concatenate([white_values, black_values], axis=0)
    table_i32 = _pack_tc(ft_weight)
    ft_out = _ft_sparsecore(table_i32, idx_all, val_all)
    ls_idx = layer_stack_indices.reshape(BATCH, 1).astype(jnp.int32)
    return _trunk_tc(ft_out, us, them, ls_idx, ft_bias,
                     W1, b1, W2, b2, Wo, bo)


# speedup vs baseline: 1.0064x; 1.0064x over previous
import functools

import jax
import jax.numpy as jnp
from jax import lax
from jax.experimental import pallas as pl
from jax.experimental.pallas import tpu as pltpu
from jax.experimental.pallas import tpu_sc as plsc

NUM_FEATURES = 22528
L1 = 1024
L2 = 15
L3 = 32
NUM_BUCKETS = 9
MAX_ACTIVE = 32
BATCH = 4096

_NC = 2
_NS = 16
_LANES = 16
_NW = _NC * _NS
_NBAGS = 2 * BATCH
_BPW = _NBAGS // _NW

_TB = 512
_L1P = 144
_WOP = 16


_DW = L1 // 2


def _ft_sparsecore(table_i32, idx_all, val_all):
    mesh = plsc.VectorSubcoreMesh(core_axis_name="c", subcore_axis_name="s")
    cp = pltpu.CompilerParams(needs_layout_passes=False)
    hb = _BPW // 2
    nbuf = 4

    @functools.partial(
        pl.kernel,
        out_type=jax.ShapeDtypeStruct((_NBAGS, L1), jnp.float32),
        mesh=mesh,
        compiler_params=cp,
        scratch_types=[
            pltpu.VMEM((hb, MAX_ACTIVE), jnp.int32),
            pltpu.VMEM((hb, MAX_ACTIVE), jnp.float32),
            pltpu.VMEM((MAX_ACTIVE, _DW), jnp.int32),
            pltpu.VMEM((MAX_ACTIVE, _DW), jnp.int32),
            pltpu.VMEM((MAX_ACTIVE, _DW), jnp.int32),
            pltpu.VMEM((MAX_ACTIVE, _DW), jnp.int32),
            pltpu.VMEM((L1,), jnp.float32),
            pltpu.VMEM((L1,), jnp.float32),
            pltpu.SemaphoreType.DMA,
            pltpu.SemaphoreType.DMA,
            pltpu.SemaphoreType.DMA,
            pltpu.SemaphoreType.DMA,
            pltpu.SemaphoreType.DMA,
            pltpu.SemaphoreType.DMA,
        ],
    )
    def ft_kernel(table_hbm, idx_hbm, val_hbm, out_hbm,
                  idx_v, val_v, rows_a, rows_b, rows_c, rows_d,
                  acc_a, acc_b,
                  gsem_a, gsem_b, gsem_c, gsem_d, osem_a, osem_b):
        wid = lax.axis_index("s") * _NC + lax.axis_index("c")
        base = wid * _BPW

        rows = (rows_a, rows_b, rows_c, rows_d)
        accs = (acc_a, acc_b)
        gsems = (gsem_a, gsem_b, gsem_c, gsem_d)
        osems = (osem_a, osem_b)

        himask = jnp.full((_LANES,), -65536, jnp.int32)

        for h in range(_BPW // hb):
            hbase = base + h * hb

            pltpu.sync_copy(idx_hbm.at[pl.ds(hbase, hb)], idx_v)
            pltpu.sync_copy(val_hbm.at[pl.ds(hbase, hb)], val_v)

            for p in range(nbuf):
                pltpu.async_copy(
                    table_hbm.at[idx_v.at[p]], rows[p], gsems[p])

            @pl.loop(0, hb // nbuf)
            def _(g):
                for par in range(nbuf):
                    bag = g * nbuf + par
                    rbuf = rows[par]
                    abuf = accs[par % 2]
                    gsem = gsems[par]
                    osem = osems[par % 2]

                    pltpu.make_async_copy(
                        table_hbm.at[idx_v.at[bag]], rbuf, gsem).wait()

                    @pl.when(h * hb + bag >= 2)
                    def _():
                        pltpu.make_async_copy(
                            abuf, out_hbm.at[hbase + bag], osem).wait()

                    row_sel = jnp.full((_LANES,), bag, jnp.int32)
                    splats = []
                    for kk in range(MAX_ACTIVE):
                        sp = plsc.load_gather(
                            val_v,
                            [row_sel, jnp.full((_LANES,), kk, jnp.int32)])
                        splats.append(
                            plsc.pack(sp, sp,
                                      format=plsc.PackFormat.INTERLEAVED))

                    @pl.loop(0, _DW, step=_LANES)
                    def _(d):
                        acc = [jnp.zeros((2 * _LANES,), jnp.bfloat16)
                               for _ in range(8)]
                        for kk in range(MAX_ACTIVE):
                            v = rbuf[kk, pl.ds(d, _LANES)]
                            q = kk % 8
                            acc[q] = acc[q] + (
                                plsc.bitcast(v, jnp.bfloat16) * splats[kk])
                        los = []
                        his = []
                        for q in range(8):
                            pi = plsc.bitcast(acc[q], jnp.int32)
                            los.append(plsc.bitcast(
                                lax.shift_left(pi, 16), jnp.float32))
                            his.append(
                                plsc.bitcast(pi & himask, jnp.float32))
                        lo = ((los[0] + los[1]) + (los[2] + los[3])) + (
                            (los[4] + los[5]) + (los[6] + los[7]))
                        hi = ((his[0] + his[1]) + (his[2] + his[3])) + (
                            (his[4] + his[5]) + (his[6] + his[7]))
                        abuf[pl.ds(d, _LANES)] = lo
                        abuf[pl.ds(_DW + d, _LANES)] = hi

                    pltpu.async_copy(abuf, out_hbm.at[hbase + bag], osem)

                    @pl.when(bag + nbuf < hb)
                    def _():
                        pltpu.async_copy(
                            table_hbm.at[idx_v.at[bag + nbuf]], rbuf, gsem)

        pltpu.make_async_copy(acc_a, out_hbm.at[base], osem_a).wait()
        pltpu.make_async_copy(acc_b, out_hbm.at[base], osem_b).wait()

    return ft_kernel(table_i32, idx_all, val_all)


def _pack_body(x_ref, o_ref):
    bits = lax.bitcast_convert_type(x_ref[...], jnp.uint32)
    r = bits + jnp.uint32(0x8000)
    word = (r[:, :_DW] >> 16) | (r[:, _DW:] & jnp.uint32(0xFFFF0000))
    o_ref[...] = lax.bitcast_convert_type(word, jnp.int32)


def _pack_tc(ft_weight):
    rb = NUM_FEATURES // 32
    return pl.pallas_call(
        _pack_body,
        grid=(32,),
        in_specs=[pl.BlockSpec((rb, L1), lambda i: (i, 0))],
        out_specs=pl.BlockSpec((rb, _DW), lambda i: (i, 0)),
        out_shape=jax.ShapeDtypeStruct((NUM_FEATURES, _DW), jnp.int32),
    )(ft_weight)


def _trunk_body(w_ref, b_ref, us_ref, them_ref, ls_ref, ftb_ref,
                w1_ref, b1_ref, w2_ref, b2_ref, wo_ref, bo_ref, o_ref):
    w = w_ref[...] + ftb_ref[...]
    b = b_ref[...] + ftb_ref[...]
    us = us_ref[...]
    them = them_ref[...]
    h1 = jnp.clip(us * w + them * b, 0.0, 1.0)
    h2 = jnp.clip(us * b + them * w, 0.0, 1.0)
    s = L1 // 2
    x = jnp.concatenate(
        [h1[:, :s] * h1[:, s:], h2[:, :s] * h2[:, s:]], axis=1)
    x = x * (127.0 / 128.0)

    idx = ls_ref[...]

    dn = (((1,), (1,)), ((), ()))
    l1 = lax.dot_general(x, w1_ref[...], dn,
                         preferred_element_type=jnp.float32) + b1_ref[...]
    c1 = lax.broadcasted_iota(jnp.int32, (_TB, _L1P), 1)
    l1 = jnp.where((c1 // L2) == idx, l1, 0.0)
    p1 = (lax.broadcasted_iota(jnp.int32, (_L1P, L2), 0) % L2
          == lax.broadcasted_iota(jnp.int32, (_L1P, L2), 1)
          ).astype(jnp.float32)
    l1 = lax.dot_general(l1, p1, (((1,), (0,)), ((), ())),
                         preferred_element_type=jnp.float32)
    l1 = jnp.clip(l1, 0.0, 1.0)

    l2 = lax.dot_general(l1, w2_ref[...], dn,
                         preferred_element_type=jnp.float32) + b2_ref[...]
    c2 = lax.broadcasted_iota(jnp.int32, (_TB, L3 * NUM_BUCKETS), 1)
    l2 = jnp.where((c2 // L3) == idx, l2, 0.0)
    p2 = (lax.broadcasted_iota(jnp.int32, (L3 * NUM_BUCKETS, L3), 0) % L3
          == lax.broadcasted_iota(jnp.int32, (L3 * NUM_BUCKETS, L3), 1)
          ).astype(jnp.float32)
    l2 = lax.dot_general(l2, p2, (((1,), (0,)), ((), ())),
                         preferred_element_type=jnp.float32)
    l2 = jnp.clip(l2, 0.0, 1.0)

    o = lax.dot_general(l2, wo_ref[...], dn,
                        preferred_element_type=jnp.float32) + bo_ref[...]
    c3 = lax.broadcasted_iota(jnp.int32, (_TB, _WOP), 1)
    o = jnp.where(c3 == idx, o, 0.0)
    o_ref[...] = jnp.sum(o, axis=1, keepdims=True)


def _trunk_tc(ft_out, us, them, ls_idx, ft_bias, W1, b1, W2, b2, Wo, bo):
    nblk = BATCH // _TB
    w1p = jnp.pad(W1, ((0, _L1P - L2 * NUM_BUCKETS), (0, 0)))
    b1p = jnp.pad(b1, (0, _L1P - L2 * NUM_BUCKETS)).reshape(1, _L1P)
    wop = jnp.pad(Wo, ((0, _WOP - NUM_BUCKETS), (0, 0)))
    bop = jnp.pad(bo, (0, _WOP - NUM_BUCKETS)).reshape(1, _WOP)

    full = lambda shape: pl.BlockSpec(shape, lambda i: (0, 0))
    return pl.pallas_call(
        _trunk_body,
        grid=(nblk,),
        in_specs=[
            pl.BlockSpec((_TB, L1), lambda i: (i, 0)),
            pl.BlockSpec((_TB, L1), lambda i: (i + nblk, 0)),
            pl.BlockSpec((_TB, 1), lambda i: (i, 0)),
            pl.BlockSpec((_TB, 1), lambda i: (i, 0)),
            pl.BlockSpec((_TB, 1), lambda i: (i, 0)),
            full((1, L1)),
            full((_L1P, L1)),
            full((1, _L1P)),
            full((L3 * NUM_BUCKETS, L2)),
            full((1, L3 * NUM_BUCKETS)),
            full((_WOP, L3)),
            full((1, _WOP)),
        ],
        out_specs=pl.BlockSpec((_TB, 1), lambda i: (i, 0)),
        out_shape=jax.ShapeDtypeStruct((BATCH, 1), jnp.float32),
    )(ft_out, ft_out, us, them, ls_idx, ft_bias.reshape(1, L1),
      w1p, b1p, W2, b2.reshape(1, L3 * NUM_BUCKETS), wop, bop)


def kernel(us, them, white_indices, white_values, black_indices, black_values,
           layer_stack_indices, ft_weight, ft_bias, W1, b1, W2, b2, Wo, bo):
    idx_all = jnp.concatenate([white_indices, black_indices], axis=0)
    val_all = jnp.concatenate([white_values, black_values], axis=0)
    table_i32 = _pack_tc(ft_weight)
    ft_out = _ft_sparsecore(table_i32, idx_all, val_all)
    ls_idx = layer_stack_indices.reshape(BATCH, 1).astype(jnp.int32)
    return _trunk_tc(ft_out, us, them, ls_idx, ft_bias,
                     W1, b1, W2, b2, Wo, bo)
